# Initial kernel scaffold; baseline (speedup 1.0000x reference)
import jax
import jax.numpy as jnp
from jax.experimental import pallas as pl


def kernel(input_ids, word_emb, pos_emb, type_emb, gamma, beta):
    raise NotImplementedError("write your pallas kernel here")


# speedup vs baseline: 1.4496x; 1.4496x over previous
import functools

import jax
import jax.numpy as jnp
from jax import lax
from jax.experimental import pallas as pl
from jax.experimental.pallas import tpu as pltpu
from jax.experimental.pallas import tpu_sc as plsc

HIDDEN = 1024
EPS = 1e-12

_NC = 2
_NS = 16
_NW = _NC * _NS

_CH = 32


def _sc_gather(idx_flat, table):
    tok = idx_flat.shape[0]
    tpw = tok // _NW
    nch = tpw // _CH
    mesh = plsc.VectorSubcoreMesh(core_axis_name="c", subcore_axis_name="s")

    @functools.partial(
        pl.kernel,
        mesh=mesh,
        out_type=jax.ShapeDtypeStruct((tok, HIDDEN), jnp.float32),
        scratch_types=[
            pltpu.VMEM((tpw,), jnp.int32),
            pltpu.VMEM((_CH, HIDDEN), jnp.float32),
            pltpu.VMEM((_CH, HIDDEN), jnp.float32),
            pltpu.SemaphoreType.DMA,
            pltpu.SemaphoreType.DMA,
            pltpu.SemaphoreType.DMA,
            pltpu.SemaphoreType.DMA,
        ],
    )
    def k(idx_hbm, table_hbm, out_hbm, idx_v, rows0, rows1, g0, g1, o0, o1):
        wid = lax.axis_index("s") * _NC + lax.axis_index("c")
        base = wid * tpw
        pltpu.sync_copy(idx_hbm.at[pl.ds(base, tpw)], idx_v)

        rows = (rows0, rows1)
        gsem = (g0, g1)
        osem = (o0, o1)

        def fire_gather(c, b):
            pltpu.async_copy(
                table_hbm.at[idx_v.at[pl.ds(c * _CH, _CH)]], rows[b], gsem[b])

        def wait_gather(c, b):
            pltpu.make_async_copy(
                table_hbm.at[idx_v.at[pl.ds(c * _CH, _CH)]], rows[b],
                gsem[b]).wait()

        def fire_out(c, b):
            pltpu.async_copy(
                rows[b], out_hbm.at[pl.ds(base + c * _CH, _CH)], osem[b])

        def wait_out(c, b):
            pltpu.make_async_copy(
                rows[b], out_hbm.at[pl.ds(base + c * _CH, _CH)],
                osem[b]).wait()

        fire_gather(0, 0)

        def body(g, carry):
            for b in range(2):
                c = g * 2 + b
                wait_gather(c, b)
                fire_out(c, b)

                @pl.when(c >= 1)
                def _():
                    wait_out(c - 1, 1 - b)

                @pl.when(c + 1 < nch)
                def _():
                    fire_gather(c + 1, 1 - b)
            return carry

        lax.fori_loop(0, nch // 2, body, 0)
        wait_out(nch - 1, (nch - 1) % 2)

    return k(idx_flat, table)


def _ln_body(x_ref, pos_ref, typ_ref, g_ref, b_ref, o_ref):
    x = x_ref[0] + pos_ref[...] + typ_ref[...]
    mean = jnp.mean(x, axis=-1, keepdims=True)
    xc = x - mean
    var = jnp.mean(xc * xc, axis=-1, keepdims=True)
    y = xc * lax.rsqrt(var + EPS)
    o_ref[0] = y * g_ref[...] + b_ref[...]


def _ln(gathered, pos_emb, type_row, gamma, beta):
    b, s, h = gathered.shape
    ts = 512
    grid = (b, s // ts)
    return pl.pallas_call(
        _ln_body,
        grid=grid,
        in_specs=[
            pl.BlockSpec((1, ts, h), lambda i, j: (i, j, 0)),
            pl.BlockSpec((ts, h), lambda i, j: (j, 0)),
            pl.BlockSpec((1, h), lambda i, j: (0, 0)),
            pl.BlockSpec((1, h), lambda i, j: (0, 0)),
            pl.BlockSpec((1, h), lambda i, j: (0, 0)),
        ],
        out_specs=pl.BlockSpec((1, ts, h), lambda i, j: (i, j, 0)),
        out_shape=jax.ShapeDtypeStruct((b, s, h), jnp.float32),
    )(gathered, pos_emb, type_row, gamma, beta)


def kernel(input_ids, word_emb, pos_emb, type_emb, gamma, beta):
    b, s = input_ids.shape
    idx = input_ids.reshape(-1).astype(jnp.int32)
    gathered = _sc_gather(idx, word_emb).reshape(b, s, HIDDEN)
    return _ln(gathered, pos_emb, type_emb[0:1],
               gamma.reshape(1, HIDDEN), beta.reshape(1, HIDDEN))


# speedup vs baseline: 1.5032x; 1.0370x over previous
import functools

import jax
import jax.numpy as jnp
from jax import lax
from jax.experimental import pallas as pl
from jax.experimental.pallas import tpu as pltpu
from jax.experimental.pallas import tpu_sc as plsc

HIDDEN = 1024
EPS = 1e-12

_NC = 2
_NS = 16
_NW = _NC * _NS

_CH = 32


def _sc_gather(idx_flat, table):
    tok = idx_flat.shape[0]
    tpw = tok // _NW
    nch = tpw // _CH
    mesh = plsc.VectorSubcoreMesh(core_axis_name="c", subcore_axis_name="s")

    @functools.partial(
        pl.kernel,
        mesh=mesh,
        out_type=jax.ShapeDtypeStruct((tok, HIDDEN), jnp.float32),
        scratch_types=[
            pltpu.VMEM((tpw,), jnp.int32),
            pltpu.VMEM((_CH, HIDDEN), jnp.float32),
            pltpu.VMEM((_CH, HIDDEN), jnp.float32),
            pltpu.SemaphoreType.DMA,
            pltpu.SemaphoreType.DMA,
            pltpu.SemaphoreType.DMA,
            pltpu.SemaphoreType.DMA,
        ],
    )
    def k(idx_hbm, table_hbm, out_hbm, idx_v, rows0, rows1, g0, g1, o0, o1):
        wid = lax.axis_index("s") * _NC + lax.axis_index("c")
        base = wid * tpw
        pltpu.sync_copy(idx_hbm.at[pl.ds(base, tpw)], idx_v)

        rows = (rows0, rows1)
        gsem = (g0, g1)
        osem = (o0, o1)

        def fire_gather(c, b):
            pltpu.async_copy(
                table_hbm.at[idx_v.at[pl.ds(c * _CH, _CH)]], rows[b], gsem[b])

        def wait_gather(c, b):
            pltpu.make_async_copy(
                table_hbm.at[idx_v.at[pl.ds(c * _CH, _CH)]], rows[b],
                gsem[b]).wait()

        def fire_out(c, b):
            pltpu.async_copy(
                rows[b], out_hbm.at[pl.ds(base + c * _CH, _CH)], osem[b])

        def wait_out(c, b):
            pltpu.make_async_copy(
                rows[b], out_hbm.at[pl.ds(base + c * _CH, _CH)],
                osem[b]).wait()

        fire_gather(0, 0)

        def body(g, carry):
            for b in range(2):
                c = g * 2 + b
                wait_gather(c, b)
                fire_out(c, b)

                @pl.when(c >= 1)
                def _():
                    wait_out(c - 1, 1 - b)

                @pl.when(c + 1 < nch)
                def _():
                    fire_gather(c + 1, 1 - b)
            return carry

        lax.fori_loop(0, nch // 2, body, 0)
        wait_out(nch - 1, (nch - 1) % 2)

    return k(idx_flat, table)


def _ln_body(x_ref, pos_ref, typ_ref, g_ref, b_ref, o_ref):
    x = x_ref[0] + pos_ref[...] + typ_ref[...]
    mean = jnp.mean(x, axis=-1, keepdims=True)
    xc = x - mean
    var = jnp.mean(xc * xc, axis=-1, keepdims=True)
    y = xc * lax.rsqrt(var + EPS)
    o_ref[0] = y * g_ref[...] + b_ref[...]


def _ln(gathered, pos_emb, type_row, gamma, beta):
    b, s, h = gathered.shape
    ts = 512
    grid = (s // ts, b)
    return pl.pallas_call(
        _ln_body,
        grid=grid,
        in_specs=[
            pl.BlockSpec((1, ts, h), lambda j, i: (i, j, 0)),
            pl.BlockSpec((ts, h), lambda j, i: (j, 0)),
            pl.BlockSpec((1, h), lambda j, i: (0, 0)),
            pl.BlockSpec((1, h), lambda j, i: (0, 0)),
            pl.BlockSpec((1, h), lambda j, i: (0, 0)),
        ],
        out_specs=pl.BlockSpec((1, ts, h), lambda j, i: (i, j, 0)),
        out_shape=jax.ShapeDtypeStruct((b, s, h), jnp.float32),
    )(gathered, pos_emb, type_row, gamma, beta)


def kernel(input_ids, word_emb, pos_emb, type_emb, gamma, beta):
    b, s = input_ids.shape
    idx = input_ids.reshape(-1).astype(jnp.int32)
    gathered = _sc_gather(idx, word_emb).reshape(b, s, HIDDEN)
    return _ln(gathered, pos_emb, type_emb[0:1],
               gamma.reshape(1, HIDDEN), beta.reshape(1, HIDDEN))


# speedup vs baseline: 1.6179x; 1.0763x over previous
import functools

import jax
import jax.numpy as jnp
from jax import lax
from jax.experimental import pallas as pl
from jax.experimental.pallas import tpu as pltpu
from jax.experimental.pallas import tpu_sc as plsc

HIDDEN = 1024
EPS = 1e-12

_NC = 2
_NS = 16
_NW = _NC * _NS

_CH = 32


def _sc_gather(idx_flat, table):
    tok = idx_flat.shape[0]
    tpw = tok // _NW
    nch = tpw // _CH
    mesh = plsc.VectorSubcoreMesh(core_axis_name="c", subcore_axis_name="s")

    @functools.partial(
        pl.kernel,
        mesh=mesh,
        out_type=jax.ShapeDtypeStruct((tok, HIDDEN), jnp.float32),
        scratch_types=[
            pltpu.VMEM((tpw,), jnp.int32),
            pltpu.VMEM((_CH, HIDDEN), jnp.float32),
            pltpu.VMEM((_CH, HIDDEN), jnp.float32),
            pltpu.SemaphoreType.DMA,
            pltpu.SemaphoreType.DMA,
            pltpu.SemaphoreType.DMA,
            pltpu.SemaphoreType.DMA,
        ],
    )
    def k(idx_hbm, table_hbm, out_hbm, idx_v, rows0, rows1, g0, g1, o0, o1):
        wid = lax.axis_index("s") * _NC + lax.axis_index("c")
        base = wid * tpw
        pltpu.sync_copy(idx_hbm.at[pl.ds(base, tpw)], idx_v)

        rows = (rows0, rows1)
        gsem = (g0, g1)
        osem = (o0, o1)

        def fire_gather(c, b):
            pltpu.async_copy(
                table_hbm.at[idx_v.at[pl.ds(c * _CH, _CH)]], rows[b], gsem[b])

        def wait_gather(c, b):
            pltpu.make_async_copy(
                table_hbm.at[idx_v.at[pl.ds(c * _CH, _CH)]], rows[b],
                gsem[b]).wait()

        def fire_out(c, b):
            pltpu.async_copy(
                rows[b], out_hbm.at[pl.ds(base + c * _CH, _CH)], osem[b])

        def wait_out(c, b):
            pltpu.make_async_copy(
                rows[b], out_hbm.at[pl.ds(base + c * _CH, _CH)],
                osem[b]).wait()

        fire_gather(0, 0)

        def body(g, carry):
            for b in range(2):
                c = g * 2 + b
                wait_gather(c, b)
                fire_out(c, b)

                @pl.when(c >= 1)
                def _():
                    wait_out(c - 1, 1 - b)

                @pl.when(c + 1 < nch)
                def _():
                    fire_gather(c + 1, 1 - b)
            return carry

        lax.fori_loop(0, nch // 2, body, 0)
        wait_out(nch - 1, (nch - 1) % 2)

    return k(idx_flat, table)


def _ln_body(x_ref, pos_ref, typ_ref, g_ref, b_ref, o_ref):
    x = x_ref[0] + pos_ref[...] + typ_ref[...]
    mean = jnp.mean(x, axis=-1, keepdims=True)
    xc = x - mean
    var = jnp.mean(xc * xc, axis=-1, keepdims=True)
    y = xc * lax.rsqrt(var + EPS)
    o_ref[0] = y * g_ref[...] + b_ref[...]


def _ln(gathered, pos_emb, type_row, gamma, beta):
    b, s, h = gathered.shape
    ts = 1024
    grid = (s // ts, b)
    return pl.pallas_call(
        _ln_body,
        grid=grid,
        in_specs=[
            pl.BlockSpec((1, ts, h), lambda j, i: (i, j, 0)),
            pl.BlockSpec((ts, h), lambda j, i: (j, 0)),
            pl.BlockSpec((1, h), lambda j, i: (0, 0)),
            pl.BlockSpec((1, h), lambda j, i: (0, 0)),
            pl.BlockSpec((1, h), lambda j, i: (0, 0)),
        ],
        out_specs=pl.BlockSpec((1, ts, h), lambda j, i: (i, j, 0)),
        out_shape=jax.ShapeDtypeStruct((b, s, h), jnp.float32),
    )(gathered, pos_emb, type_row, gamma, beta)


def kernel(input_ids, word_emb, pos_emb, type_emb, gamma, beta):
    b, s = input_ids.shape
    idx = input_ids.reshape(-1).astype(jnp.int32)
    gathered = _sc_gather(idx, word_emb).reshape(b, s, HIDDEN)
    return _ln(gathered, pos_emb, type_emb[0:1],
               gamma.reshape(1, HIDDEN), beta.reshape(1, HIDDEN))


# speedup vs baseline: 1.6551x; 1.0230x over previous
import functools

import jax
import jax.numpy as jnp
from jax import lax
from jax.experimental import pallas as pl
from jax.experimental.pallas import tpu as pltpu
from jax.experimental.pallas import tpu_sc as plsc

HIDDEN = 1024
EPS = 1e-12

_NC = 2
_NS = 16
_NW = _NC * _NS

_CH = 32


def _sc_gather(idx_flat, table):
    tok = idx_flat.shape[0]
    tpw = tok // _NW
    nch = tpw // _CH
    mesh = plsc.VectorSubcoreMesh(core_axis_name="c", subcore_axis_name="s")

    @functools.partial(
        pl.kernel,
        mesh=mesh,
        out_type=jax.ShapeDtypeStruct((tok, HIDDEN), jnp.float32),
        scratch_types=[
            pltpu.VMEM((tpw,), jnp.int32),
            pltpu.VMEM((_CH, HIDDEN), jnp.float32),
            pltpu.VMEM((_CH, HIDDEN), jnp.float32),
            pltpu.SemaphoreType.DMA,
            pltpu.SemaphoreType.DMA,
            pltpu.SemaphoreType.DMA,
            pltpu.SemaphoreType.DMA,
        ],
    )
    def k(idx_hbm, table_hbm, out_hbm, idx_v, rows0, rows1, g0, g1, o0, o1):
        wid = lax.axis_index("s") * _NC + lax.axis_index("c")
        base = wid * tpw
        pltpu.sync_copy(idx_hbm.at[pl.ds(base, tpw)], idx_v)

        rows = (rows0, rows1)
        gsem = (g0, g1)
        osem = (o0, o1)

        def fire_gather(c, b):
            pltpu.async_copy(
                table_hbm.at[idx_v.at[pl.ds(c * _CH, _CH)]], rows[b], gsem[b])

        def wait_gather(c, b):
            pltpu.make_async_copy(
                table_hbm.at[idx_v.at[pl.ds(c * _CH, _CH)]], rows[b],
                gsem[b]).wait()

        def fire_out(c, b):
            pltpu.async_copy(
                rows[b], out_hbm.at[pl.ds(base + c * _CH, _CH)], osem[b])

        def wait_out(c, b):
            pltpu.make_async_copy(
                rows[b], out_hbm.at[pl.ds(base + c * _CH, _CH)],
                osem[b]).wait()

        fire_gather(0, 0)

        def body(g, carry):
            for b in range(2):
                c = g * 2 + b
                wait_gather(c, b)
                fire_out(c, b)

                @pl.when(c >= 1)
                def _():
                    wait_out(c - 1, 1 - b)

                @pl.when(c + 1 < nch)
                def _():
                    fire_gather(c + 1, 1 - b)
            return carry

        lax.fori_loop(0, nch // 2, body, 0)
        wait_out(nch - 1, (nch - 1) % 2)

    return k(idx_flat, table)


def _ln_body(x_ref, pos_ref, typ_ref, g_ref, b_ref, o_ref):
    x = x_ref[0] + pos_ref[...] + typ_ref[...]
    mean = jnp.mean(x, axis=-1, keepdims=True)
    xc = x - mean
    var = jnp.mean(xc * xc, axis=-1, keepdims=True)
    y = xc * lax.rsqrt(var + EPS)
    o_ref[0] = y * g_ref[...] + b_ref[...]


def _ln(gathered, pos_emb, type_row, gamma, beta):
    b, s, h = gathered.shape
    ts = 2048
    grid = (s // ts, b)
    return pl.pallas_call(
        _ln_body,
        grid=grid,
        in_specs=[
            pl.BlockSpec((1, ts, h), lambda j, i: (i, j, 0)),
            pl.BlockSpec((ts, h), lambda j, i: (j, 0)),
            pl.BlockSpec((1, h), lambda j, i: (0, 0)),
            pl.BlockSpec((1, h), lambda j, i: (0, 0)),
            pl.BlockSpec((1, h), lambda j, i: (0, 0)),
        ],
        out_specs=pl.BlockSpec((1, ts, h), lambda j, i: (i, j, 0)),
        out_shape=jax.ShapeDtypeStruct((b, s, h), jnp.float32),
    )(gathered, pos_emb, type_row, gamma, beta)


def kernel(input_ids, word_emb, pos_emb, type_emb, gamma, beta):
    b, s = input_ids.shape
    idx = input_ids.reshape(-1).astype(jnp.int32)
    gathered = _sc_gather(idx, word_emb).reshape(b, s, HIDDEN)
    return _ln(gathered, pos_emb, type_emb[0:1],
               gamma.reshape(1, HIDDEN), beta.reshape(1, HIDDEN))
